# Initial kernel scaffold; baseline (speedup 1.0000x reference)
import jax
import jax.numpy as jnp
from jax.experimental import pallas as pl


def kernel(x, table):
    raise NotImplementedError("write your pallas kernel here")


# speedup vs baseline: 1.6839x; 1.6839x over previous
import functools

import jax
import jax.numpy as jnp
from jax import lax
from jax.experimental import pallas as pl
from jax.experimental.pallas import tpu as pltpu
from jax.experimental.pallas import tpu_sc as plsc

CHUNK = 128


@functools.lru_cache(maxsize=None)
def _make_gather(vocab: int, dim: int, batch: int):
    info = plsc.get_sparse_core_info()
    nc, ns = info.num_cores, info.num_subcores
    nw = nc * ns
    n_per_w = batch // nw
    n_chunks = n_per_w // CHUNK
    assert n_per_w % CHUNK == 0

    mesh = plsc.VectorSubcoreMesh(core_axis_name="c", subcore_axis_name="s")

    @functools.partial(
        pl.kernel,
        mesh=mesh,
        out_type=jax.ShapeDtypeStruct((batch, dim), jnp.float32),
        scratch_types=[
            pltpu.VMEM((n_chunks, CHUNK), jnp.int32),
            pltpu.VMEM((CHUNK, dim), jnp.float32),
            pltpu.SemaphoreType.DMA,
        ],
        compiler_params=pltpu.CompilerParams(use_tc_tiling_on_sc=False),
    )
    def gather_kernel(x_hbm, table_hbm, out_hbm, idx_v, rows_v, sem):
        wid = lax.axis_index("s") * nc + lax.axis_index("c")
        base = wid * n_per_w
        pltpu.sync_copy(x_hbm.at[wid], idx_v)

        def step(j, carry):
            pltpu.async_copy(table_hbm.at[idx_v.at[j]], rows_v, sem).wait()
            pltpu.sync_copy(rows_v, out_hbm.at[pl.ds(base + j * CHUNK, CHUNK)])
            return carry

        lax.fori_loop(0, n_chunks, step, 0)

    return gather_kernel, nw, n_chunks


def kernel(x, table):
    vocab, dim = table.shape
    batch = x.size
    gather_kernel, nw, n_chunks = _make_gather(vocab, dim, batch)
    xf = x.astype(jnp.int32).reshape(nw, n_chunks, CHUNK)
    out = gather_kernel(xf, table)
    return out.reshape(*x.shape, dim)


# speedup vs baseline: 1.8712x; 1.1112x over previous
import functools

import jax
import jax.numpy as jnp
from jax import lax
from jax.experimental import pallas as pl
from jax.experimental.pallas import tpu as pltpu
from jax.experimental.pallas import tpu_sc as plsc

CHUNK = 128
K = 4


@functools.lru_cache(maxsize=None)
def _make_gather(vocab: int, dim: int, batch: int):
    info = plsc.get_sparse_core_info()
    nc, ns = info.num_cores, info.num_subcores
    nw = nc * ns
    n_per_w = batch // nw
    n_chunks = n_per_w // CHUNK
    n_groups = n_chunks // K
    assert batch == nw * n_chunks * CHUNK and n_chunks % K == 0

    mesh = plsc.VectorSubcoreMesh(core_axis_name="c", subcore_axis_name="s")

    @functools.partial(
        pl.kernel,
        mesh=mesh,
        out_type=jax.ShapeDtypeStruct((batch, dim), jnp.float32),
        scratch_types=[
            pltpu.VMEM((n_chunks, CHUNK), jnp.int32),
            pltpu.VMEM((2, K, CHUNK, dim), jnp.float32),
            pltpu.SemaphoreType.DMA,
            pltpu.SemaphoreType.DMA,
        ],
        compiler_params=pltpu.CompilerParams(use_tc_tiling_on_sc=False),
    )
    def gather_kernel(x_hbm, table_hbm, out_hbm, idx_v, rows_v, gsem, osem):
        wid = lax.axis_index("s") * nc + lax.axis_index("c")
        base = wid * n_per_w
        pltpu.sync_copy(x_hbm.at[wid], idx_v)

        def group(g, carry):
            p = lax.rem(g, 2)

            @pl.when(g >= 2)
            def _():
                for k in range(K):
                    pltpu.make_async_copy(
                        rows_v.at[p, k], out_hbm.at[pl.ds(base, CHUNK)], osem
                    ).wait()

            copies = []
            for k in range(K):
                j = g * K + k
                copies.append(
                    pltpu.async_copy(
                        table_hbm.at[idx_v.at[j]], rows_v.at[p, k], gsem
                    )
                )
            for c in copies:
                c.wait()
            for k in range(K):
                j = g * K + k
                pltpu.make_async_copy(
                    rows_v.at[p, k],
                    out_hbm.at[pl.ds(base + j * CHUNK, CHUNK)],
                    osem,
                ).start()
            return carry

        lax.fori_loop(0, n_groups, group, 0)

        for _ in range(2 * K):
            pltpu.make_async_copy(
                rows_v.at[0, 0], out_hbm.at[pl.ds(base, CHUNK)], osem
            ).wait()

    return gather_kernel, nw, n_chunks


def kernel(x, table):
    vocab, dim = table.shape
    batch = x.size
    gather_kernel, nw, n_chunks = _make_gather(vocab, dim, batch)
    xf = x.astype(jnp.int32).reshape(nw, n_chunks, CHUNK)
    out = gather_kernel(xf, table)
    return out.reshape(*x.shape, dim)
